# Initial kernel scaffold; baseline (speedup 1.0000x reference)
import jax
import jax.numpy as jnp
from jax.experimental import pallas as pl


def kernel(x, scalars, node_mask, params):
    raise NotImplementedError("write your pallas kernel here")


# speedup vs baseline: 7.5301x; 7.5301x over previous
import functools

import jax
import jax.numpy as jnp
from jax import lax
from jax.experimental import pallas as pl

NH = 128
NSC = 8
CW = 0.001
F32 = jnp.float32
METRIC = (1.0, -1.0, -1.0, -1.0)


def _psi(t):
    return jnp.sign(t) * jnp.log(jnp.abs(t) + 1.0)


def _metric_vec():
    lane = lax.broadcasted_iota(jnp.int32, (1, 4), 1)
    return jnp.where(lane == 0, 1.0, -1.0).astype(F32)


def _mink_pairs(xi, xa):
    mv = _metric_vec()
    xmi = xi * mv
    dots = lax.dot_general(xmi, xa, (((1,), (1,)), ((), ())),
                           preferred_element_type=F32)
    ni = jnp.sum(xi * xmi, axis=1)
    na = jnp.sum(xa * (xa * mv), axis=1)
    norms = ni[:, None] + na[None, :] - 2.0 * dots
    return _psi(norms), _psi(dots)


def _embed_kernel(scal_ref, ew_ref, eb_ref, wab_ref, h_ref, ha_ref, hb_ref):
    s = scal_ref[...]
    h = jnp.dot(s, ew_ref[...], preferred_element_type=F32) + eb_ref[...]
    h_ref[...] = h
    wab = wab_ref[...]
    ha_ref[...] = jnp.dot(h, wab[:NH], preferred_element_type=F32)
    hb_ref[...] = jnp.dot(h, wab[NH:], preferred_element_type=F32)


def _stats_kernel(x_ref, ha_ref, hb_ref, wnd_ref, acc_ref, *, n, nch):
    b = pl.program_id(0)
    x = x_ref[0]
    hb = hb_ref[0]
    wn = wnd_ref[0:1].reshape(1, 1, NH)
    wd = wnd_ref[1:2].reshape(1, 1, NH)
    ch = n // nch
    s = jnp.zeros((NH,), F32)
    sq = jnp.zeros((NH,), F32)
    for c in range(nch):
        xi = x_ref[0, c * ch:(c + 1) * ch, :]
        pn, pd = _mink_pairs(xi, x)
        hac = ha_ref[0, c * ch:(c + 1) * ch, :]
        m1 = (hac[:, None, :] + hb[None, :, :]
              + pn[..., None] * wn + pd[..., None] * wd)
        jj = lax.broadcasted_iota(jnp.int32, (ch, n, NH), 1)
        ig = c * ch + lax.broadcasted_iota(jnp.int32, (ch, n, NH), 0)
        m1m = jnp.where(jj != ig, m1, 0.0)
        s = s + jnp.sum(m1m, axis=(0, 1))
        sq = sq + jnp.sum(m1m * m1, axis=(0, 1))
    rows = lax.broadcasted_iota(jnp.int32, (8, NH), 0)
    val = jnp.where(rows == 0, s[None, :],
                    jnp.where(rows == 1, sq[None, :], 0.0))

    @pl.when(b == 0)
    def _():
        acc_ref[...] = val

    @pl.when(b > 0)
    def _():
        acc_ref[...] = acc_ref[...] + val


def _edge_kernel(xi_ref, xa_ref, ha_ref, hb_ref, wnd_ref, acc_ref, ebn_ref,
                 e2w_ref, e2b_ref, mwt_ref, mb_ref, x1w_ref, x1b_ref,
                 x2wt_ref, aggm_ref, aggx_ref, *, n, bi, r_edges, has_x):
    ib = pl.program_id(1)
    xi = xi_ref[0]
    xa = xa_ref[0]
    ha = ha_ref[0]
    hb = hb_ref[0]
    wn = wnd_ref[0:1].reshape(1, 1, NH)
    wd = wnd_ref[1:2].reshape(1, 1, NH)

    mu = acc_ref[0:1] * (1.0 / r_edges)
    var = acc_ref[1:2] * (1.0 / r_edges) - mu * mu
    scale = ebn_ref[0:1] * lax.rsqrt(var + 1e-5)
    shift = ebn_ref[1:2] - mu * scale

    pn, pd = _mink_pairs(xi, xa)
    m1 = (ha[:, None, :] + hb[None, :, :]
          + pn[..., None] * wn + pd[..., None] * wd)
    m2 = jax.nn.relu(m1 * scale[None] + shift[None]).reshape(bi * n, NH)
    m3 = jax.nn.relu(jnp.dot(m2, e2w_ref[...], preferred_element_type=F32)
                     + e2b_ref[...])
    wl = jnp.sum(m3 * mwt_ref[...], axis=1) + mb_ref[0, 0]
    mg = m3 * jax.nn.sigmoid(wl)[:, None]

    jj = lax.broadcasted_iota(jnp.int32, (bi, n, NH), 1)
    ig = ib * bi + lax.broadcasted_iota(jnp.int32, (bi, n, NH), 0)
    mg3 = jnp.where(jj != ig, mg.reshape(bi, n, NH), 0.0)
    aggm_ref[0] = jnp.sum(mg3, axis=1)

    if has_x:
        t1 = jax.nn.relu(jnp.dot(mg, x1w_ref[...], preferred_element_type=F32)
                         + x1b_ref[...])
        t = jnp.sum(t1 * x2wt_ref[...], axis=1).reshape(bi, n, 1)
        xd = xi[:, None, :] - xa[None, :, :]
        trans = jnp.clip(xd * t, -100.0, 100.0)
        aggx_ref[0] = jnp.sum(trans, axis=1)
    else:
        aggx_ref[0] = jnp.zeros((bi, 4), F32)


def _node_kernel(h_ref, am_ref, scal_ref, x_ref, ax_ref, wh_ref, wa_ref,
                 ws_ref, h1b_ref, hbn_ref, h2w_ref, h2b_ref, wab_ref,
                 ho_ref, xo_ref, hao_ref, hbo_ref, *, n, has_next, has_x):
    h = h_ref[...]
    o1 = (jnp.dot(h, wh_ref[...], preferred_element_type=F32)
          + jnp.dot(am_ref[...], wa_ref[...], preferred_element_type=F32)
          + jnp.dot(scal_ref[...], ws_ref[...], preferred_element_type=F32)
          + h1b_ref[...])
    mu = jnp.mean(o1, axis=0, keepdims=True)
    var = jnp.mean(o1 * o1, axis=0, keepdims=True) - mu * mu
    g = hbn_ref[0:1] * lax.rsqrt(var + 1e-5)
    o = jax.nn.relu((o1 - mu) * g + hbn_ref[1:2])
    hn = h + jnp.dot(o, h2w_ref[...], preferred_element_type=F32) + h2b_ref[...]
    ho_ref[...] = hn
    if has_x:
        xo_ref[...] = x_ref[...] + ax_ref[...] * (CW / (n - 1.0))
    else:
        xo_ref[...] = x_ref[...]
    if has_next:
        wab = wab_ref[...]
        hao_ref[...] = jnp.dot(hn, wab[:NH], preferred_element_type=F32)
        hbo_ref[...] = jnp.dot(hn, wab[NH:], preferred_element_type=F32)


def _final_kernel(h_ref, mask_ref, d1w_ref, d1b_ref, d2w_ref, d2b_ref,
                  out_ref, *, n):
    hm = jnp.sum(h_ref[...] * mask_ref[...], axis=1) * (1.0 / n)
    z = jax.nn.relu(jnp.dot(hm, d1w_ref[...], preferred_element_type=F32)
                    + d1b_ref[...])
    out_ref[...] = (jnp.dot(z, d2w_ref[...], preferred_element_type=F32)
                    + d2b_ref[...])


def _full(shape):
    nd = len(shape)
    return pl.BlockSpec(shape, lambda *_: (0,) * nd)


def kernel(x, scalars, node_mask, params):
    b, n, _ = x.shape
    bn = b * n
    nblk = 4
    bi = n // nblk
    r_edges = float(b * n * (n - 1))

    def row(v):
        return v.reshape(1, -1).astype(F32)

    emb_w = params["emb"]["W"].astype(F32)
    emb_b = row(params["emb"]["b"])
    layers = params["layers"]
    wab0 = layers[0]["e1"]["W"][:2 * NH].astype(F32)

    scal2 = scalars.reshape(bn, NSC).astype(F32)
    h, ha, hb = pl.pallas_call(
        _embed_kernel,
        out_shape=[jax.ShapeDtypeStruct((bn, NH), F32)] * 3,
        in_specs=[_full((bn, NSC)), _full((NSC, NH)), _full((1, NH)),
                  _full((2 * NH, NH))],
        out_specs=[_full((bn, NH))] * 3,
    )(scal2, emb_w, emb_b, wab0)

    xc = x.astype(F32)
    for li, lp in enumerate(layers):
        has_x = "x1" in lp
        has_next = li + 1 < len(layers)
        wnd = lp["e1"]["W"][2 * NH:].astype(F32)
        ebn = jnp.stack([lp["e_bn"]["g"], lp["e_bn"]["b"]]).astype(F32)
        hbn = jnp.stack([lp["h_bn"]["g"], lp["h_bn"]["b"]]).astype(F32)
        ha3 = ha.reshape(b, n, NH)
        hb3 = hb.reshape(b, n, NH)

        acc = pl.pallas_call(
            functools.partial(_stats_kernel, n=n, nch=4),
            grid=(b,),
            out_shape=jax.ShapeDtypeStruct((8, NH), F32),
            in_specs=[
                pl.BlockSpec((1, n, 4), lambda bb: (bb, 0, 0)),
                pl.BlockSpec((1, n, NH), lambda bb: (bb, 0, 0)),
                pl.BlockSpec((1, n, NH), lambda bb: (bb, 0, 0)),
                pl.BlockSpec((2, NH), lambda bb: (0, 0)),
            ],
            out_specs=pl.BlockSpec((8, NH), lambda bb: (0, 0)),
        )(xc, ha3, hb3, wnd)

        mwt = row(lp["m"]["W"])
        mb = lp["m"]["b"].reshape(1, 1).astype(F32)
        if has_x:
            x1w = lp["x1"]["W"].astype(F32)
            x1b = row(lp["x1"]["b"])
            x2wt = row(lp["x2"]["W"])
        else:
            x1w = jnp.zeros((NH, NH), F32)
            x1b = jnp.zeros((1, NH), F32)
            x2wt = jnp.zeros((1, NH), F32)

        aggm, aggx = pl.pallas_call(
            functools.partial(_edge_kernel, n=n, bi=bi, r_edges=r_edges,
                              has_x=has_x),
            grid=(b, nblk),
            out_shape=[jax.ShapeDtypeStruct((b, n, NH), F32),
                       jax.ShapeDtypeStruct((b, n, 4), F32)],
            in_specs=[
                pl.BlockSpec((1, bi, 4), lambda bb, ib: (bb, ib, 0)),
                pl.BlockSpec((1, n, 4), lambda bb, ib: (bb, 0, 0)),
                pl.BlockSpec((1, bi, NH), lambda bb, ib: (bb, ib, 0)),
                pl.BlockSpec((1, n, NH), lambda bb, ib: (bb, 0, 0)),
                pl.BlockSpec((2, NH), lambda bb, ib: (0, 0)),
                pl.BlockSpec((8, NH), lambda bb, ib: (0, 0)),
                pl.BlockSpec((2, NH), lambda bb, ib: (0, 0)),
                pl.BlockSpec((NH, NH), lambda bb, ib: (0, 0)),
                pl.BlockSpec((1, NH), lambda bb, ib: (0, 0)),
                pl.BlockSpec((1, NH), lambda bb, ib: (0, 0)),
                pl.BlockSpec((1, 1), lambda bb, ib: (0, 0)),
                pl.BlockSpec((NH, NH), lambda bb, ib: (0, 0)),
                pl.BlockSpec((1, NH), lambda bb, ib: (0, 0)),
                pl.BlockSpec((1, NH), lambda bb, ib: (0, 0)),
            ],
            out_specs=[pl.BlockSpec((1, bi, NH), lambda bb, ib: (bb, ib, 0)),
                       pl.BlockSpec((1, bi, 4), lambda bb, ib: (bb, ib, 0))],
        )(xc, xc, ha3, hb3, wnd, acc, ebn, lp["e2"]["W"].astype(F32),
          row(lp["e2"]["b"]), mwt, mb, x1w, x1b, x2wt)

        h1w = lp["h1"]["W"].astype(F32)
        wab_next = (layers[li + 1]["e1"]["W"][:2 * NH].astype(F32)
                    if has_next else jnp.zeros((2 * NH, NH), F32))
        h, xc, ha, hb = pl.pallas_call(
            functools.partial(_node_kernel, n=n, has_next=has_next,
                              has_x=has_x),
            out_shape=[jax.ShapeDtypeStruct((bn, NH), F32),
                       jax.ShapeDtypeStruct((b, n, 4), F32),
                       jax.ShapeDtypeStruct((bn, NH), F32),
                       jax.ShapeDtypeStruct((bn, NH), F32)],
            in_specs=[_full((bn, NH)), _full((bn, NH)), _full((bn, NSC)),
                      _full((b, n, 4)), _full((b, n, 4)),
                      _full((NH, NH)), _full((NH, NH)), _full((NSC, NH)),
                      _full((1, NH)), _full((2, NH)), _full((NH, NH)),
                      _full((1, NH)), _full((2 * NH, NH))],
            out_specs=[_full((bn, NH)), _full((b, n, 4)),
                       _full((bn, NH)), _full((bn, NH))],
        )(h, aggm.reshape(bn, NH), scal2, xc, aggx,
          h1w[:NH], h1w[NH:2 * NH], h1w[2 * NH:], row(lp["h1"]["b"]), hbn,
          lp["h2"]["W"].astype(F32), row(lp["h2"]["b"]), wab_next)

    pred = pl.pallas_call(
        functools.partial(_final_kernel, n=n),
        out_shape=jax.ShapeDtypeStruct((b, 2), F32),
        in_specs=[_full((b, n, NH)), _full((b, n, 1)), _full((NH, NH)),
                  _full((1, NH)), _full((NH, 2)), _full((1, 2))],
        out_specs=_full((b, 2)),
    )(h.reshape(b, n, NH), node_mask.astype(F32), params["d1"]["W"].astype(F32),
      row(params["d1"]["b"]), params["d2"]["W"].astype(F32),
      row(params["d2"]["b"]))
    return pred
